# baseline (speedup 1.0000x reference)
import functools

import jax
import jax.numpy as jnp
import numpy as np
from jax import lax
from jax.experimental import pallas as pl
from jax.experimental.pallas import tpu as pltpu
from jax.experimental.pallas import tpu_sc as plsc

NC = 2
NS = 16
NW = NC * NS
LANES = 16

CAND = 128
CAND_PAD = 160
POOL = 512
I32_MIN = np.int32(-(2 ** 31))
NEG_BIG = np.float32(-3.0e38)


def _key_from_bits(b):
    return jnp.where(b >= 0, b, I32_MIN - b)


def _sload(ref, i):
    return ref[pl.ds(i, LANES)][0]


def _butterfly(x, op):
    parts = [x[i] for i in range(LANES)]
    while len(parts) > 1:
        parts = [op(parts[i], parts[i + 1])
                 for i in range(0, len(parts), 2)]
    return parts[0]


def _allsum(x):
    return _butterfly(x, jnp.add)


def _allmax(x):
    return _butterfly(x, jnp.maximum)


def _allmin(x):
    return _butterfly(x, jnp.minimum)


def _sc_body(V, ROWS, kb_hbm, k_hbm, p_hbm, qf_hbm, q2d_hbm, out_hbm,
             keys_v, qbuf_v, pool_v, qi_v, q_v, kall_v, pall_v, outb_v,
             ck_sm, ci_sm, probs_sm, sem):
    B = ROWS * NW
    wid = lax.axis_index("s") * NC + lax.axis_index("c")
    base = wid * ROWS
    lane = lax.iota(jnp.int32, 16)
    NCH = V // LANES

    pltpu.sync_copy(k_hbm, kall_v.at[pl.ds(0, B)])
    pltpu.sync_copy(p_hbm, pall_v.at[pl.ds(0, B)])

    def row_body(j, winners):
        r = base + j
        pltpu.sync_copy(kb_hbm.at[r], keys_v)

        def conv_body(i, c):
            off = i * LANES
            b = keys_v[pl.ds(off, LANES)]
            keys_v[pl.ds(off, LANES)] = _key_from_bits(b)
            return c
        lax.fori_loop(0, NCH, conv_body, 0, unroll=8)

        kk = _sload(kall_v, r)
        prow = _sload(pall_v, r)

        def count_ge(t):
            def cbody(i, acc):
                kv = keys_v[pl.ds(i * LANES, LANES)]
                return acc + jnp.where(kv >= t, 1, 0)
            acc = lax.fori_loop(0, NCH, cbody, jnp.zeros((16,), jnp.int32),
                                unroll=8)
            return _allsum(acc)

        c0 = count_ge(jnp.int32(0))
        tbase = jnp.where(c0 >= kk, jnp.int32(0), I32_MIN)

        def bit_body(i, acc_bits):
            bit = jnp.int32(1) << (jnp.int32(30) - i)
            cand_t = tbase | acc_bits | bit
            cnt_t = count_ge(cand_t)
            return jnp.where(cnt_t >= kk, acc_bits | bit, acc_bits)
        accA = lax.fori_loop(0, 12, bit_body, jnp.int32(0))
        t12 = tbase | accA

        def pbody(i, off):
            kv = keys_v[pl.ds(i * LANES, LANES)]
            hit = kv >= t12
            hi = jnp.where(hit, 1, 0)
            pc = plsc.cumsum(hi)
            pos = jnp.where(hit,
                            jnp.minimum(off + pc - 1, POOL - 1),
                            POOL + lane)
            plsc.store_scatter(pool_v, [pos], kv)
            return off + pc[15]
        poolcnt = lax.fori_loop(0, NCH, pbody, jnp.int32(0), unroll=4)

        def fast_finish(_):
            def bitp(i, acc):
                bit = jnp.int32(1) << (jnp.int32(18) - i)
                cand_t = t12 | acc | bit

                def cb(ii, a):
                    pv = pool_v[pl.ds(ii * LANES, LANES)]
                    ok = (pv >= cand_t) & ((ii * LANES + lane) < poolcnt)
                    return a + jnp.where(ok, 1, 0)
                cnt_t = _allsum(lax.fori_loop(
                    0, POOL // LANES, cb, jnp.zeros((16,), jnp.int32)))
                return jnp.where(cnt_t >= kk, acc | bit, acc)
            return t12 | lax.fori_loop(0, 19, bitp, jnp.int32(0))

        def slow_finish(_):
            def bit_body2(i, acc_bits):
                bit = jnp.int32(1) << (jnp.int32(30) - i)
                cand_t = tbase | accA | acc_bits | bit
                cnt_t = count_ge(cand_t)
                return jnp.where(cnt_t >= kk, acc_bits | bit, acc_bits)
            return t12 | lax.fori_loop(12, 31, bit_body2, jnp.int32(0))

        thresh = lax.cond(poolcnt <= POOL, fast_finish, slow_finish, 0)

        def ebody(i, off):
            kv = keys_v[pl.ds(i * LANES, LANES)]
            m = kv >= thresh
            anyhit = jnp.any(m)

            def take(o):
                cb = i * LANES
                for l in range(LANES):
                    ck_sm[o] = kv[l]
                    ci_sm[o] = cb + l
                    hit = (kv[l] >= thresh).astype(jnp.int32)
                    o = jnp.minimum(o + hit, CAND_PAD - 1)
                return o

            return lax.cond(anyhit, take, lambda o: o, off)
        cnt_true = lax.fori_loop(0, NCH, ebody, jnp.int32(0), unroll=2)
        cnt = jnp.minimum(cnt_true, jnp.int32(CAND))
        rowoff = r * V

        QCH = V // 10
        QIN = QCH // LANES

        def qscan_c(c, zs):
            pltpu.sync_copy(q2d_hbm.at[r * 10 + c], qbuf_v)

            def qscan_i(i2, zs2):
                qv = qbuf_v[pl.ds(i2 * LANES, LANES)]
                hasz = jnp.any(qv == 0.0)

                def hit(zz):
                    z1, z2 = zz
                    cb = c * QCH + i2 * LANES
                    for l in range(LANES):
                        is0 = qv[l] == 0.0
                        gidx = jnp.int32(cb + l)
                        first = is0 & (z1 >= V)
                        second = is0 & (z1 < V) & (z2 >= V) & (gidx != z1)
                        z1 = jnp.where(first, gidx, z1)
                        z2 = jnp.where(second, gidx, z2)
                    return z1, z2

                return lax.cond(hasz, hit, lambda zz: zz, zs2)
            return lax.fori_loop(0, QIN, qscan_i, zs)
        zq1, zq2 = lax.fori_loop(0, 10, qscan_c,
                                 (jnp.int32(V), jnp.int32(V)))

        NSL = CAND // LANES
        cks, cis, cifs, valids, vs = [], [], [], [], []
        mvec = jnp.full((16,), NEG_BIG, jnp.float32)
        for s in range(NSL):
            ck = jnp.zeros((16,), jnp.int32)
            ci = jnp.zeros((16,), jnp.int32)
            for l in range(LANES):
                ck = jnp.where(lane == l, ck_sm[s * LANES + l], ck)
                ci = jnp.where(lane == l, ci_sm[s * LANES + l], ci)
            slot = lane + s * LANES
            valid = slot < cnt
            v = lax.bitcast_convert_type(_key_from_bits(ck), jnp.float32)
            cks.append(ck); cis.append(ci); valids.append(valid); vs.append(v)
            cifs.append(ci.astype(jnp.float32))
            mvec = jnp.maximum(mvec, jnp.where(valid, v, NEG_BIG))
            cic = jnp.minimum(jnp.maximum(ci, 0), V - 1)
            qi_v[pl.ds(s * LANES, LANES)] = cic + rowoff
        pltpu.async_copy(qf_hbm.at[qi_v], q_v, sem).wait()
        m = _allmax(mvec)

        es = []
        zacc = jnp.zeros((16,), jnp.float32)
        for s in range(NSL):
            e = jnp.where(valids[s], jnp.exp(vs[s] - m), 0.0)
            es.append(e)
            zacc = zacc + e
        Z0 = _allsum(zacc)
        pvs = []
        for s in range(NSL):
            pv = es[s] / Z0
            pvs.append(pv)
            for l in range(LANES):
                probs_sm[s * LANES + l] = pv[l]

        imaxv = jnp.full((16,), -1.0, jnp.float32)
        for s in range(NSL):
            imaxv = jnp.maximum(
                imaxv, jnp.where(valids[s] & (vs[s] == m), cifs[s], -1.0))
        imax = _allmax(imaxv)

        def jbody(jj, caccs):
            kj = ck_sm[jj]
            ij = ci_sm[jj]
            pj = probs_sm[jj]
            out = []
            for s in range(NSL):
                le = (kj < cks[s]) | ((kj == cks[s]) & (ij <= cis[s]))
                out.append(caccs[s] + jnp.where(le, pj, 0.0))
            return tuple(out)
        caccs = lax.fori_loop(
            0, cnt, jbody,
            tuple(jnp.zeros((16,), jnp.float32) for _ in range(NSL)))

        omp = jnp.float32(1.0) - prow
        keeps = []
        z2acc = jnp.zeros((16,), jnp.float32)
        for s in range(NSL):
            keep = valids[s] & ((caccs[s] > omp)
                                | ((vs[s] == m) & (cifs[s] == imax)))
            keeps.append(keep)
            z2acc = z2acc + jnp.where(keep, es[s], 0.0)
        Z2 = _allsum(z2acc)

        rs, qs = [], []
        rmaxv = jnp.zeros((16,), jnp.float32)
        for s in range(NSL):
            qv = q_v[pl.ds(s * LANES, LANES)]
            rr = jnp.where(keeps[s], (es[s] / Z2) / qv, 0.0)
            rs.append(rr)
            qs.append(qv)
            rmaxv = jnp.maximum(rmaxv, rr)
        rmax = _allmax(rmaxv)
        wv = jnp.full((16,), float(V), jnp.float32)
        nanv = jnp.full((16,), float(V), jnp.float32)
        for s in range(NSL):
            wv = jnp.minimum(
                wv, jnp.where(keeps[s] & (rs[s] == rmax), cifs[s], float(V)))
            nanv = jnp.minimum(
                nanv, jnp.where(valids[s] & (~keeps[s]) & (qs[s] == 0.0),
                                cifs[s], float(V)))
        winner = _allmin(wv).astype(jnp.int32)
        mem1v = jnp.zeros((16,), jnp.int32)
        mem2v = jnp.zeros((16,), jnp.int32)
        for s in range(NSL):
            mem1v = jnp.maximum(
                mem1v, jnp.where(valids[s] & (cis[s] == zq1), 1, 0))
            mem2v = jnp.maximum(
                mem2v, jnp.where(valids[s] & (cis[s] == zq2), 1, 0))
        mem1 = _allmax(mem1v) > 0
        mem2 = _allmax(mem2v) > 0
        zmin_noncand = jnp.where(
            (zq1 < V) & (~mem1), zq1,
            jnp.where((zq2 < V) & (~mem2), zq2, jnp.int32(V)))
        nanw = jnp.minimum(_allmin(nanv).astype(jnp.int32), zmin_noncand)
        winner = jnp.where(nanw < V, nanw, winner)

        return jnp.where(lane == j, winner, winners)

    winners = lax.fori_loop(0, ROWS, row_body, jnp.zeros((16,), jnp.int32))
    outb_v[...] = winners
    pltpu.sync_copy(outb_v, out_hbm.at[wid])


@functools.partial(jax.jit, static_argnums=(4, 5))
def _run(kb, kvec, pvec, qf, V, ROWS):
    body = functools.partial(_sc_body, V, ROWS)
    f = pl.kernel(
        body,
        out_type=jax.ShapeDtypeStruct((NW, 16), jnp.int32),
        mesh=plsc.VectorSubcoreMesh(core_axis_name="c", subcore_axis_name="s"),
        compiler_params=pltpu.CompilerParams(needs_layout_passes=False),
        scratch_types=[
            pltpu.VMEM((V,), jnp.int32),
            pltpu.VMEM((V // 10,), jnp.float32),
            pltpu.VMEM((POOL + LANES,), jnp.int32),
            pltpu.VMEM((CAND,), jnp.int32),
            pltpu.VMEM((CAND,), jnp.float32),
            pltpu.VMEM((NW * ROWS + LANES,), jnp.int32),
            pltpu.VMEM((NW * ROWS + LANES,), jnp.float32),
            pltpu.VMEM((16,), jnp.int32),
            pltpu.SMEM((CAND_PAD,), jnp.int32),
            pltpu.SMEM((CAND_PAD,), jnp.int32),
            pltpu.SMEM((CAND,), jnp.float32),
            pltpu.SemaphoreType.DMA,
        ],
    )
    return f(kb, kvec, pvec, qf, qf.reshape(-1, V // 10))


def kernel(logits, k, p, q):
    B, V = logits.shape
    ROWS = B // NW
    kb = lax.bitcast_convert_type(logits.astype(jnp.float32), jnp.int32)
    out = _run(kb, k.astype(jnp.int32), p.astype(jnp.float32),
               q.astype(jnp.float32).reshape(-1), V, ROWS)
    return out[:, :ROWS].reshape(B)


# speedup vs baseline: 1.5761x; 1.5761x over previous
import functools

import jax
import jax.numpy as jnp
import numpy as np
from jax import lax
from jax.experimental import pallas as pl
from jax.experimental.pallas import tpu as pltpu
from jax.experimental.pallas import tpu_sc as plsc

NC = 2
NS = 16
NW = NC * NS
LANES = 16

CAND = 128
CAND_PAD = 160
POOL = 512
I32_MIN = np.int32(-(2 ** 31))
NEG_BIG = np.float32(-3.0e38)


def _key_from_bits(b):
    return jnp.where(b >= 0, b, I32_MIN - b)


def _sload(ref, i):
    return ref[pl.ds(i, LANES)][0]


def _butterfly(x, op):
    parts = [x[i] for i in range(LANES)]
    while len(parts) > 1:
        parts = [op(parts[i], parts[i + 1])
                 for i in range(0, len(parts), 2)]
    return parts[0]


def _allsum(x):
    return _butterfly(x, jnp.add)


def _allmax(x):
    return _butterfly(x, jnp.maximum)


def _allmin(x):
    return _butterfly(x, jnp.minimum)


def _sc_body(V, ROWS, kb_hbm, k_hbm, p_hbm, qf_hbm, q2d_hbm, out_hbm,
             keys_v, qbuf_v, pool_v, ck_v, ci_v, probs_v, qi_v, q_v,
             kall_v, pall_v, outb_v, sem):
    B = ROWS * NW
    wid = lax.axis_index("s") * NC + lax.axis_index("c")
    base = wid * ROWS
    lane = lax.iota(jnp.int32, 16)
    NCH = V // LANES

    pltpu.sync_copy(k_hbm, kall_v.at[pl.ds(0, B)])
    pltpu.sync_copy(p_hbm, pall_v.at[pl.ds(0, B)])

    def row_body(j, winners):
        r = base + j
        pltpu.sync_copy(kb_hbm.at[r], keys_v)

        def conv_body(i, c):
            off = i * LANES
            b = keys_v[pl.ds(off, LANES)]
            keys_v[pl.ds(off, LANES)] = _key_from_bits(b)
            return c
        lax.fori_loop(0, NCH, conv_body, 0, unroll=8)

        kk = _sload(kall_v, r)
        prow = _sload(pall_v, r)

        def count_ge(t):
            def cbody(i, acc):
                kv = keys_v[pl.ds(i * LANES, LANES)]
                return acc + jnp.where(kv >= t, 1, 0)
            acc = lax.fori_loop(0, NCH, cbody, jnp.zeros((16,), jnp.int32),
                                unroll=8)
            return _allsum(acc)

        c0 = count_ge(jnp.int32(0))
        tbase = jnp.where(c0 >= kk, jnp.int32(0), I32_MIN)

        def bit_body(i, acc_bits):
            bit = jnp.int32(1) << (jnp.int32(30) - i)
            cand_t = tbase | acc_bits | bit
            cnt_t = count_ge(cand_t)
            return jnp.where(cnt_t >= kk, acc_bits | bit, acc_bits)
        accA = lax.fori_loop(0, 12, bit_body, jnp.int32(0))
        t12 = tbase | accA

        def pbody(i, off):
            kv = keys_v[pl.ds(i * LANES, LANES)]
            hit = kv >= t12
            hi = jnp.where(hit, 1, 0)
            pc = plsc.cumsum(hi)
            pos = jnp.where(hit,
                            jnp.minimum(off + pc - 1, POOL - 1),
                            POOL + lane)
            plsc.store_scatter(pool_v, [pos], kv)
            return off + pc[15]
        poolcnt = lax.fori_loop(0, NCH, pbody, jnp.int32(0), unroll=4)

        def fast_finish(_):
            def bitp(i, acc):
                bit = jnp.int32(1) << (jnp.int32(18) - i)
                cand_t = t12 | acc | bit

                def cb(ii, a):
                    pv = pool_v[pl.ds(ii * LANES, LANES)]
                    ok = (pv >= cand_t) & ((ii * LANES + lane) < poolcnt)
                    return a + jnp.where(ok, 1, 0)
                cnt_t = _allsum(lax.fori_loop(
                    0, POOL // LANES, cb, jnp.zeros((16,), jnp.int32)))
                return jnp.where(cnt_t >= kk, acc | bit, acc)
            return t12 | lax.fori_loop(0, 19, bitp, jnp.int32(0))

        def slow_finish(_):
            def bit_body2(i, acc_bits):
                bit = jnp.int32(1) << (jnp.int32(30) - i)
                cand_t = tbase | accA | acc_bits | bit
                cnt_t = count_ge(cand_t)
                return jnp.where(cnt_t >= kk, acc_bits | bit, acc_bits)
            return t12 | lax.fori_loop(12, 31, bit_body2, jnp.int32(0))

        thresh = lax.cond(poolcnt <= POOL, fast_finish, slow_finish, 0)

        def ebody(i, off):
            kv = keys_v[pl.ds(i * LANES, LANES)]
            hit = kv >= thresh
            pc = plsc.cumsum(jnp.where(hit, 1, 0))
            pos = jnp.where(hit,
                            jnp.minimum(off + pc - 1, CAND_PAD - 1),
                            CAND_PAD + lane)
            plsc.store_scatter(ck_v, [pos], kv)
            plsc.store_scatter(ci_v, [pos], lane + i * LANES)
            return off + pc[15]
        cnt_true = lax.fori_loop(0, NCH, ebody, jnp.int32(0), unroll=4)
        cnt = jnp.minimum(cnt_true, jnp.int32(CAND))
        rowoff = r * V

        QCH = V // 10
        QIN = QCH // LANES

        def qscan_c(c, zs):
            z1, z2, zc = zs
            pltpu.sync_copy(q2d_hbm.at[r * 10 + c], qbuf_v)

            def count_z(i2, acc):
                qv = qbuf_v[pl.ds(i2 * LANES, LANES)]
                return acc + jnp.where(qv == 0.0, 1, 0)
            zacc0 = lax.fori_loop(0, QIN, count_z,
                                  jnp.zeros((16,), jnp.int32), unroll=8)
            nz = _allsum(zacc0)

            def find(zz):
                z1, z2 = zz

                def fbody(i2, zz2):
                    z1, z2 = zz2
                    qv = qbuf_v[pl.ds(i2 * LANES, LANES)]
                    for l in range(LANES):
                        is0 = qv[l] == 0.0
                        gidx = c * QCH + i2 * LANES + l
                        first = is0 & (z1 >= V)
                        second = is0 & (z1 < V) & (z2 >= V) & (gidx != z1)
                        z1 = jnp.where(first, gidx, z1)
                        z2 = jnp.where(second, gidx, z2)
                    return z1, z2
                return lax.fori_loop(0, QIN, fbody, (z1, z2))

            z1, z2 = lax.cond(nz > 0, find, lambda zz: zz, (z1, z2))
            return z1, z2, zc + nz
        zq1, zq2, _ = lax.fori_loop(
            0, 10, qscan_c, (jnp.int32(V), jnp.int32(V), jnp.int32(0)))

        NSL = CAND // LANES
        cks, cis, cifs, valids, vs = [], [], [], [], []
        mvec = jnp.full((16,), NEG_BIG, jnp.float32)
        for s in range(NSL):
            ck = ck_v[pl.ds(s * LANES, LANES)]
            ci = ci_v[pl.ds(s * LANES, LANES)]
            slot = lane + s * LANES
            valid = slot < cnt
            v = lax.bitcast_convert_type(_key_from_bits(ck), jnp.float32)
            cks.append(ck); cis.append(ci); valids.append(valid); vs.append(v)
            cifs.append(ci.astype(jnp.float32))
            mvec = jnp.maximum(mvec, jnp.where(valid, v, NEG_BIG))
            cic = jnp.minimum(jnp.maximum(ci, 0), V - 1)
            qi_v[pl.ds(s * LANES, LANES)] = cic + rowoff
        pltpu.async_copy(qf_hbm.at[qi_v], q_v, sem).wait()
        m = _allmax(mvec)

        es = []
        zacc = jnp.zeros((16,), jnp.float32)
        for s in range(NSL):
            e = jnp.where(valids[s], jnp.exp(vs[s] - m), 0.0)
            es.append(e)
            zacc = zacc + e
        Z0 = _allsum(zacc)
        pvs = []
        for s in range(NSL):
            pv = es[s] / Z0
            pvs.append(pv)
            probs_v[pl.ds(s * LANES, LANES)] = pv

        imaxv = jnp.full((16,), -1.0, jnp.float32)
        for s in range(NSL):
            imaxv = jnp.maximum(
                imaxv, jnp.where(valids[s] & (vs[s] == m), cifs[s], -1.0))
        imax = _allmax(imaxv)

        def jbody(jj, caccs):
            kj = _sload(ck_v, jj)
            ij = _sload(ci_v, jj)
            pj = _sload(probs_v, jj)
            out = []
            for s in range(NSL):
                le = (kj < cks[s]) | ((kj == cks[s]) & (ij <= cis[s]))
                out.append(caccs[s] + jnp.where(le, pj, 0.0))
            return tuple(out)
        caccs = lax.fori_loop(
            0, cnt, jbody,
            tuple(jnp.zeros((16,), jnp.float32) for _ in range(NSL)))

        omp = jnp.float32(1.0) - prow
        keeps = []
        z2acc = jnp.zeros((16,), jnp.float32)
        for s in range(NSL):
            keep = valids[s] & ((caccs[s] > omp)
                                | ((vs[s] == m) & (cifs[s] == imax)))
            keeps.append(keep)
            z2acc = z2acc + jnp.where(keep, es[s], 0.0)
        Z2 = _allsum(z2acc)

        rs, qs = [], []
        rmaxv = jnp.zeros((16,), jnp.float32)
        for s in range(NSL):
            qv = q_v[pl.ds(s * LANES, LANES)]
            rr = jnp.where(keeps[s], (es[s] / Z2) / qv, 0.0)
            rs.append(rr)
            qs.append(qv)
            rmaxv = jnp.maximum(rmaxv, rr)
        rmax = _allmax(rmaxv)
        wv = jnp.full((16,), float(V), jnp.float32)
        nanv = jnp.full((16,), float(V), jnp.float32)
        for s in range(NSL):
            wv = jnp.minimum(
                wv, jnp.where(keeps[s] & (rs[s] == rmax), cifs[s], float(V)))
            nanv = jnp.minimum(
                nanv, jnp.where(valids[s] & (~keeps[s]) & (qs[s] == 0.0),
                                cifs[s], float(V)))
        winner = _allmin(wv).astype(jnp.int32)
        mem1v = jnp.zeros((16,), jnp.int32)
        mem2v = jnp.zeros((16,), jnp.int32)
        for s in range(NSL):
            mem1v = jnp.maximum(
                mem1v, jnp.where(valids[s] & (cis[s] == zq1), 1, 0))
            mem2v = jnp.maximum(
                mem2v, jnp.where(valids[s] & (cis[s] == zq2), 1, 0))
        mem1 = _allmax(mem1v) > 0
        mem2 = _allmax(mem2v) > 0
        zmin_noncand = jnp.where(
            (zq1 < V) & (~mem1), zq1,
            jnp.where((zq2 < V) & (~mem2), zq2, jnp.int32(V)))
        nanw = jnp.minimum(_allmin(nanv).astype(jnp.int32), zmin_noncand)
        winner = jnp.where(nanw < V, nanw, winner)

        return jnp.where(lane == j, winner, winners)

    winners = lax.fori_loop(0, ROWS, row_body, jnp.zeros((16,), jnp.int32))
    outb_v[...] = winners
    pltpu.sync_copy(outb_v, out_hbm.at[wid])


@functools.partial(jax.jit, static_argnums=(4, 5))
def _run(kb, kvec, pvec, qf, V, ROWS):
    body = functools.partial(_sc_body, V, ROWS)
    f = pl.kernel(
        body,
        out_type=jax.ShapeDtypeStruct((NW, 16), jnp.int32),
        mesh=plsc.VectorSubcoreMesh(core_axis_name="c", subcore_axis_name="s"),
        compiler_params=pltpu.CompilerParams(needs_layout_passes=False),
        scratch_types=[
            pltpu.VMEM((V,), jnp.int32),
            pltpu.VMEM((V // 10,), jnp.float32),
            pltpu.VMEM((POOL + LANES,), jnp.int32),
            pltpu.VMEM((CAND_PAD + LANES,), jnp.int32),
            pltpu.VMEM((CAND_PAD + LANES,), jnp.int32),
            pltpu.VMEM((CAND + LANES,), jnp.float32),
            pltpu.VMEM((CAND,), jnp.int32),
            pltpu.VMEM((CAND,), jnp.float32),
            pltpu.VMEM((NW * ROWS + LANES,), jnp.int32),
            pltpu.VMEM((NW * ROWS + LANES,), jnp.float32),
            pltpu.VMEM((16,), jnp.int32),
            pltpu.SemaphoreType.DMA,
        ],
    )
    return f(kb, kvec, pvec, qf, qf.reshape(-1, V // 10))


def kernel(logits, k, p, q):
    B, V = logits.shape
    ROWS = B // NW
    kb = lax.bitcast_convert_type(logits.astype(jnp.float32), jnp.int32)
    out = _run(kb, k.astype(jnp.int32), p.astype(jnp.float32),
               q.astype(jnp.float32).reshape(-1), V, ROWS)
    return out[:, :ROWS].reshape(B)


# speedup vs baseline: 2.0938x; 1.3285x over previous
import functools

import jax
import jax.numpy as jnp
import numpy as np
from jax import lax
from jax.experimental import pallas as pl
from jax.experimental.pallas import tpu as pltpu
from jax.experimental.pallas import tpu_sc as plsc

NC = 2
NS = 16
NW = NC * NS
LANES = 16

CAND = 128
CAND_PAD = 160
POOL = 512
I32_MIN = np.int32(-(2 ** 31))
NEG_BIG = np.float32(-3.0e38)


def _key_from_bits(b):
    return jnp.where(b >= 0, b, I32_MIN - b)


def _sload(ref, i):
    return ref[pl.ds(i, LANES)][0]


def _butterfly(x, op):
    parts = [x[i] for i in range(LANES)]
    while len(parts) > 1:
        parts = [op(parts[i], parts[i + 1])
                 for i in range(0, len(parts), 2)]
    return parts[0]


def _allsum(x):
    return _butterfly(x, jnp.add)


def _allmax(x):
    return _butterfly(x, jnp.maximum)


def _allmin(x):
    return _butterfly(x, jnp.minimum)


def _sc_body(V, ROWS, kb_hbm, k_hbm, p_hbm, qf_hbm, q2d_hbm, out_hbm,
             keys_v, qbuf_v, pool_v, pool_i_v, ck_v, ci_v, probs_v, qi_v,
             q_v, kall_v, pall_v, outb_v, sem):
    B = ROWS * NW
    wid = lax.axis_index("s") * NC + lax.axis_index("c")
    base = wid * ROWS
    lane = lax.iota(jnp.int32, 16)
    NCH = V // LANES

    pltpu.sync_copy(k_hbm, kall_v.at[pl.ds(0, B)])
    pltpu.sync_copy(p_hbm, pall_v.at[pl.ds(0, B)])

    def row_body(j, winners):
        r = base + j
        pltpu.sync_copy(kb_hbm.at[r], keys_v)

        def conv_body(i, c):
            off = i * LANES
            b = keys_v[pl.ds(off, LANES)]
            keys_v[pl.ds(off, LANES)] = _key_from_bits(b)
            return c
        lax.fori_loop(0, NCH, conv_body, 0, unroll=8)

        kk = _sload(kall_v, r)
        prow = _sload(pall_v, r)

        def count_ge(t):
            def cbody(i, acc):
                kv = keys_v[pl.ds(i * LANES, LANES)]
                return acc + jnp.where(kv >= t, 1, 0)
            acc = lax.fori_loop(0, NCH, cbody, jnp.zeros((16,), jnp.int32),
                                unroll=8)
            return _allsum(acc)

        c0 = count_ge(jnp.int32(0))
        tbase = jnp.where(c0 >= kk, jnp.int32(0), I32_MIN)

        def bit_body(i, acc_bits):
            bit = jnp.int32(1) << (jnp.int32(30) - i)
            cand_t = tbase | acc_bits | bit
            cnt_t = count_ge(cand_t)
            return jnp.where(cnt_t >= kk, acc_bits | bit, acc_bits)
        accA = lax.fori_loop(0, 12, bit_body, jnp.int32(0))
        t12 = tbase | accA

        def pool_fill(t):
            def pbody(i, off):
                kv = keys_v[pl.ds(i * LANES, LANES)]
                hit = kv >= t
                hi = jnp.where(hit, 1, 0)
                pc = plsc.cumsum(hi)
                pos = jnp.where(hit,
                                jnp.minimum(off + pc - 1, POOL - 1),
                                POOL + lane)
                plsc.store_scatter(pool_v, [pos], kv)
                plsc.store_scatter(pool_i_v, [pos], lane + i * LANES)
                return off + pc[15]
            return lax.fori_loop(0, NCH, pbody, jnp.int32(0), unroll=4)
        poolcnt = pool_fill(t12)

        def fast_finish(_):
            def bitp(i, acc):
                bit = jnp.int32(1) << (jnp.int32(18) - i)
                cand_t = t12 | acc | bit

                def cb(ii, a):
                    pv = pool_v[pl.ds(ii * LANES, LANES)]
                    ok = (pv >= cand_t) & ((ii * LANES + lane) < poolcnt)
                    return a + jnp.where(ok, 1, 0)
                cnt_t = _allsum(lax.fori_loop(
                    0, POOL // LANES, cb, jnp.zeros((16,), jnp.int32)))
                return jnp.where(cnt_t >= kk, acc | bit, acc)
            return t12 | lax.fori_loop(0, 19, bitp, jnp.int32(0))

        def slow_finish(_):
            def bit_body2(i, acc_bits):
                bit = jnp.int32(1) << (jnp.int32(30) - i)
                cand_t = tbase | accA | acc_bits | bit
                cnt_t = count_ge(cand_t)
                return jnp.where(cnt_t >= kk, acc_bits | bit, acc_bits)
            return t12 | lax.fori_loop(12, 31, bit_body2, jnp.int32(0))

        thresh = lax.cond(poolcnt <= POOL, fast_finish, slow_finish, 0)
        poolcnt = lax.cond(poolcnt <= POOL, lambda t: poolcnt, pool_fill,
                           thresh)

        def ebody(i, off):
            kv = pool_v[pl.ds(i * LANES, LANES)]
            iv = pool_i_v[pl.ds(i * LANES, LANES)]
            hit = (kv >= thresh) & ((lane + i * LANES) < poolcnt)
            pc = plsc.cumsum(jnp.where(hit, 1, 0))
            pos = jnp.where(hit,
                            jnp.minimum(off + pc - 1, CAND_PAD - 1),
                            CAND_PAD + lane)
            plsc.store_scatter(ck_v, [pos], kv)
            plsc.store_scatter(ci_v, [pos], iv)
            return off + pc[15]
        cnt_true = lax.fori_loop(0, POOL // LANES, ebody, jnp.int32(0),
                                 unroll=4)
        cnt = jnp.minimum(cnt_true, jnp.int32(CAND))
        rowoff = r * V

        QCH = V // 10
        QIN = QCH // LANES

        def qscan_c(c, zs):
            z1, z2, zc = zs
            pltpu.sync_copy(q2d_hbm.at[r * 10 + c], qbuf_v)

            def count_z(i2, acc):
                qv = qbuf_v[pl.ds(i2 * LANES, LANES)]
                return acc + jnp.where(qv == 0.0, 1, 0)
            zacc0 = lax.fori_loop(0, QIN, count_z,
                                  jnp.zeros((16,), jnp.int32), unroll=8)
            nz = _allsum(zacc0)

            def find(zz):
                z1, z2 = zz

                def fbody(i2, zz2):
                    z1, z2 = zz2
                    qv = qbuf_v[pl.ds(i2 * LANES, LANES)]
                    for l in range(LANES):
                        is0 = qv[l] == 0.0
                        gidx = c * QCH + i2 * LANES + l
                        first = is0 & (z1 >= V)
                        second = is0 & (z1 < V) & (z2 >= V) & (gidx != z1)
                        z1 = jnp.where(first, gidx, z1)
                        z2 = jnp.where(second, gidx, z2)
                    return z1, z2
                return lax.fori_loop(0, QIN, fbody, (z1, z2))

            z1, z2 = lax.cond(nz > 0, find, lambda zz: zz, (z1, z2))
            return z1, z2, zc + nz
        zq1, zq2, _ = lax.fori_loop(
            0, 10, qscan_c, (jnp.int32(V), jnp.int32(V), jnp.int32(0)))

        NSL = CAND // LANES
        cks, cis, cifs, valids, vs = [], [], [], [], []
        mvec = jnp.full((16,), NEG_BIG, jnp.float32)
        for s in range(NSL):
            ck = ck_v[pl.ds(s * LANES, LANES)]
            ci = ci_v[pl.ds(s * LANES, LANES)]
            slot = lane + s * LANES
            valid = slot < cnt
            v = lax.bitcast_convert_type(_key_from_bits(ck), jnp.float32)
            cks.append(ck); cis.append(ci); valids.append(valid); vs.append(v)
            cifs.append(ci.astype(jnp.float32))
            mvec = jnp.maximum(mvec, jnp.where(valid, v, NEG_BIG))
            cic = jnp.minimum(jnp.maximum(ci, 0), V - 1)
            qi_v[pl.ds(s * LANES, LANES)] = cic + rowoff
        pltpu.async_copy(qf_hbm.at[qi_v], q_v, sem).wait()
        m = _allmax(mvec)

        es = []
        zacc = jnp.zeros((16,), jnp.float32)
        for s in range(NSL):
            e = jnp.where(valids[s], jnp.exp(vs[s] - m), 0.0)
            es.append(e)
            zacc = zacc + e
        Z0 = _allsum(zacc)
        pvs = []
        for s in range(NSL):
            pv = es[s] / Z0
            pvs.append(pv)
            probs_v[pl.ds(s * LANES, LANES)] = pv

        imaxv = jnp.full((16,), -1.0, jnp.float32)
        for s in range(NSL):
            imaxv = jnp.maximum(
                imaxv, jnp.where(valids[s] & (vs[s] == m), cifs[s], -1.0))
        imax = _allmax(imaxv)

        def jbody(jj, caccs):
            kj = _sload(ck_v, jj)
            ij = _sload(ci_v, jj)
            pj = _sload(probs_v, jj)
            out = []
            for s in range(NSL):
                le = (kj < cks[s]) | ((kj == cks[s]) & (ij <= cis[s]))
                out.append(caccs[s] + jnp.where(le, pj, 0.0))
            return tuple(out)
        caccs = lax.fori_loop(
            0, cnt, jbody,
            tuple(jnp.zeros((16,), jnp.float32) for _ in range(NSL)))

        omp = jnp.float32(1.0) - prow
        keeps = []
        z2acc = jnp.zeros((16,), jnp.float32)
        for s in range(NSL):
            keep = valids[s] & ((caccs[s] > omp)
                                | ((vs[s] == m) & (cifs[s] == imax)))
            keeps.append(keep)
            z2acc = z2acc + jnp.where(keep, es[s], 0.0)
        Z2 = _allsum(z2acc)

        rs, qs = [], []
        rmaxv = jnp.zeros((16,), jnp.float32)
        for s in range(NSL):
            qv = q_v[pl.ds(s * LANES, LANES)]
            rr = jnp.where(keeps[s], (es[s] / Z2) / qv, 0.0)
            rs.append(rr)
            qs.append(qv)
            rmaxv = jnp.maximum(rmaxv, rr)
        rmax = _allmax(rmaxv)
        wv = jnp.full((16,), float(V), jnp.float32)
        nanv = jnp.full((16,), float(V), jnp.float32)
        for s in range(NSL):
            wv = jnp.minimum(
                wv, jnp.where(keeps[s] & (rs[s] == rmax), cifs[s], float(V)))
            nanv = jnp.minimum(
                nanv, jnp.where(valids[s] & (~keeps[s]) & (qs[s] == 0.0),
                                cifs[s], float(V)))
        winner = _allmin(wv).astype(jnp.int32)
        mem1v = jnp.zeros((16,), jnp.int32)
        mem2v = jnp.zeros((16,), jnp.int32)
        for s in range(NSL):
            mem1v = jnp.maximum(
                mem1v, jnp.where(valids[s] & (cis[s] == zq1), 1, 0))
            mem2v = jnp.maximum(
                mem2v, jnp.where(valids[s] & (cis[s] == zq2), 1, 0))
        mem1 = _allmax(mem1v) > 0
        mem2 = _allmax(mem2v) > 0
        zmin_noncand = jnp.where(
            (zq1 < V) & (~mem1), zq1,
            jnp.where((zq2 < V) & (~mem2), zq2, jnp.int32(V)))
        nanw = jnp.minimum(_allmin(nanv).astype(jnp.int32), zmin_noncand)
        winner = jnp.where(nanw < V, nanw, winner)

        return jnp.where(lane == j, winner, winners)

    winners = lax.fori_loop(0, ROWS, row_body, jnp.zeros((16,), jnp.int32))
    outb_v[...] = winners
    pltpu.sync_copy(outb_v, out_hbm.at[wid])


@functools.partial(jax.jit, static_argnums=(4, 5))
def _run(kb, kvec, pvec, qf, V, ROWS):
    body = functools.partial(_sc_body, V, ROWS)
    f = pl.kernel(
        body,
        out_type=jax.ShapeDtypeStruct((NW, 16), jnp.int32),
        mesh=plsc.VectorSubcoreMesh(core_axis_name="c", subcore_axis_name="s"),
        compiler_params=pltpu.CompilerParams(needs_layout_passes=False),
        scratch_types=[
            pltpu.VMEM((V,), jnp.int32),
            pltpu.VMEM((V // 10,), jnp.float32),
            pltpu.VMEM((POOL + LANES,), jnp.int32),
            pltpu.VMEM((POOL + LANES,), jnp.int32),
            pltpu.VMEM((CAND_PAD + LANES,), jnp.int32),
            pltpu.VMEM((CAND_PAD + LANES,), jnp.int32),
            pltpu.VMEM((CAND + LANES,), jnp.float32),
            pltpu.VMEM((CAND,), jnp.int32),
            pltpu.VMEM((CAND,), jnp.float32),
            pltpu.VMEM((NW * ROWS + LANES,), jnp.int32),
            pltpu.VMEM((NW * ROWS + LANES,), jnp.float32),
            pltpu.VMEM((16,), jnp.int32),
            pltpu.SemaphoreType.DMA,
        ],
    )
    return f(kb, kvec, pvec, qf, qf.reshape(-1, V // 10))


def kernel(logits, k, p, q):
    B, V = logits.shape
    ROWS = B // NW
    kb = lax.bitcast_convert_type(logits.astype(jnp.float32), jnp.int32)
    out = _run(kb, k.astype(jnp.int32), p.astype(jnp.float32),
               q.astype(jnp.float32).reshape(-1), V, ROWS)
    return out[:, :ROWS].reshape(B)
